# Initial kernel scaffold; baseline (speedup 1.0000x reference)
import jax
import jax.numpy as jnp
from jax.experimental import pallas as pl


def kernel(z, obs_t, mask_t, params, consts):
    raise NotImplementedError("write your pallas kernel here")


# speedup vs baseline: 1.9065x; 1.9065x over previous
import functools
import math

import jax
import jax.numpy as jnp
from jax import lax
from jax.experimental import pallas as pl
from jax.experimental.pallas import tpu as pltpu

B = 4096
LAT = 64
OBS = 72
HID = 1024
J = 29
NCMD = 16
K = 8
TOPK = 2
GH = 256
POS = 75
EH = 512
LOG_STD_MIN = math.log(1e-4)
LOG_STD_MAX = math.log(5.0)

_F32 = jnp.float32


def _elu(x):
    return jnp.where(x > 0, x, jnp.exp(jnp.minimum(x, 0.0)) - 1.0)


def _softmax(x):
    m = jnp.max(x, axis=-1, keepdims=True)
    e = jnp.exp(x - m)
    return e / jnp.sum(e, axis=-1, keepdims=True)


def _ln(x, g, b):
    m = x.mean(-1, keepdims=True)
    v = ((x - m) ** 2).mean(-1, keepdims=True)
    return (x - m) * jax.lax.rsqrt(v + 1e-5) * g + b


def _gate_body(obs_ref, z_ref, chW1, chb1, chW2, chb2, gW1, gb1, gW2, gb2,
               gW3, gb3, lows, highs, cmdn_out, wn_out):
    obs = obs_ref[...]
    z = z_ref[...]
    oz = jnp.concatenate([obs, z], axis=-1)
    h = _elu(jnp.dot(oz, chW1[...], preferred_element_type=_F32) + chb1[...])
    cmd01 = jax.nn.sigmoid(jnp.dot(h, chW2[...], preferred_element_type=_F32) + chb2[...])
    lo = lows[...]
    cmd = lo + (highs[...] - lo) * cmd01
    g_in = jnp.concatenate([cmd, obs[:, NCMD:], z], axis=-1)
    g = _elu(jnp.dot(g_in, gW1[...], preferred_element_type=_F32) + gb1[...])
    g = _elu(jnp.dot(g, gW2[...], preferred_element_type=_F32) + gb2[...])
    logits = jnp.dot(g, gW3[...], preferred_element_type=_F32) + gb3[...]
    w = _softmax(logits)
    kidx = lax.broadcasted_iota(jnp.int32, w.shape, 1)
    m1 = jnp.max(w, axis=-1, keepdims=True)
    i1 = jnp.min(jnp.where(w == m1, kidx, K), axis=-1, keepdims=True)
    w2 = jnp.where(kidx == i1, -1.0, w)
    m2 = jnp.max(w2, axis=-1, keepdims=True)
    i2 = jnp.min(jnp.where(w2 == m2, kidx, K), axis=-1, keepdims=True)
    msk = (kidx == i1) | (kidx == i2)
    wm = jnp.where(msk, w, 0.0)
    wn = wm / jnp.sum(wm, axis=-1, keepdims=True)
    cmdn_out[...] = cmd
    wn_out[...] = wn


def _expert_body(obs_ref, cmdn_ref, wn_ref, W1f, b1f, W2, b2, W3, b3, out_ref):
    x = jnp.concatenate([cmdn_ref[...], obs_ref[:, NCMD:]], axis=-1)
    h1 = _elu(jnp.dot(x, W1f[...], preferred_element_type=_F32) + b1f[...])
    wn = wn_ref[...]
    acc = jnp.zeros((x.shape[0], J), _F32)
    for k in range(K):
        h1k = h1[:, k * EH:(k + 1) * EH]
        h2 = _elu(jnp.dot(h1k, W2[k], preferred_element_type=_F32) + b2[k:k + 1, :])
        mu = jnp.dot(h2, W3[k], preferred_element_type=_F32) + b3[k:k + 1, :]
        acc = acc + wn[:, k:k + 1] * mu
    out_ref[...] = acc


def _fn_body(obs_ref, z_ref, mask_ref, fnW1, fnb1, g1, be1, fnW2, fnb2, g2,
             be2, lsW1, lsb1, lsW2, lsb2, vhW, vhb,
             feats_out, ls_out, sig_out):
    oz = jnp.concatenate([obs_ref[...], z_ref[...]], axis=-1)
    x = jnp.dot(oz, fnW1[...], preferred_element_type=_F32) + fnb1[...]
    x = jax.nn.relu(_ln(x, g1[...], be1[...]))
    x = jnp.dot(x, fnW2[...], preferred_element_type=_F32) + fnb2[...]
    x = jax.nn.relu(_ln(x, g2[...], be2[...]))
    feats = x * mask_ref[...]
    h = jax.nn.relu(jnp.dot(feats, lsW1[...], preferred_element_type=_F32) + lsb1[...])
    ls = jnp.dot(h, lsW2[...], preferred_element_type=_F32) + lsb2[...]
    log_std = jnp.clip(ls, LOG_STD_MIN, LOG_STD_MAX)
    sr = jnp.dot(feats, vhW[...], preferred_element_type=_F32) + vhb[...]
    sigma = 0.05 + (0.5 - 0.05) * jax.nn.sigmoid(sr)
    feats_out[...] = feats
    ls_out[...] = log_std
    sig_out[...] = jnp.log(sigma)


def _row_spec(t, n):
    return pl.BlockSpec((t, n), lambda i: (0, 0) if t is None else (i, 0))


def _full_spec(shape):
    nd = len(shape)
    return pl.BlockSpec(shape, lambda i, _nd=nd: (0,) * _nd)


def kernel(z, obs_t, mask_t, params, consts):
    p, c = params, consts
    r2 = lambda a: a.reshape(1, -1)

    TG = 512
    cmdn, wn = pl.pallas_call(
        _gate_body,
        grid=(B // TG,),
        in_specs=[
            pl.BlockSpec((TG, OBS), lambda i: (i, 0)),
            pl.BlockSpec((TG, LAT), lambda i: (i, 0)),
            _full_spec((OBS + LAT, GH)), _full_spec((1, GH)),
            _full_spec((GH, NCMD)), _full_spec((1, NCMD)),
            _full_spec((OBS + LAT, GH)), _full_spec((1, GH)),
            _full_spec((GH, GH)), _full_spec((1, GH)),
            _full_spec((GH, K)), _full_spec((1, K)),
            _full_spec((1, NCMD)), _full_spec((1, NCMD)),
        ],
        out_specs=[
            pl.BlockSpec((TG, NCMD), lambda i: (i, 0)),
            pl.BlockSpec((TG, K), lambda i: (i, 0)),
        ],
        out_shape=[
            jax.ShapeDtypeStruct((B, NCMD), _F32),
            jax.ShapeDtypeStruct((B, K), _F32),
        ],
        compiler_params=pltpu.CompilerParams(
            dimension_semantics=("arbitrary",)),
    )(obs_t, z, p['ch_W1'], r2(p['ch_b1']), p['ch_W2'], r2(p['ch_b2']),
      p['g_W1'], r2(p['g_b1']), p['g_W2'], r2(p['g_b2']),
      p['g_W3'], r2(p['g_b3']), r2(c['cmd_lows']), r2(c['cmd_highs']))

    W1f = (c['ex_W1'] * jnp.concatenate(
        [c['cmd_masks'][:, :, None],
         jnp.ones((K, OBS - NCMD, 1), _F32)], axis=1))
    W1f = W1f.transpose(1, 0, 2).reshape(OBS, K * EH)
    b1f = c['ex_b1'].reshape(1, K * EH)

    TE = 512
    mu_mix = pl.pallas_call(
        _expert_body,
        grid=(B // TE,),
        in_specs=[
            pl.BlockSpec((TE, OBS), lambda i: (i, 0)),
            pl.BlockSpec((TE, NCMD), lambda i: (i, 0)),
            pl.BlockSpec((TE, K), lambda i: (i, 0)),
            _full_spec((OBS, K * EH)), _full_spec((1, K * EH)),
            _full_spec((K, EH, EH)), _full_spec((K, EH)),
            _full_spec((K, EH, J)), _full_spec((K, J)),
        ],
        out_specs=[pl.BlockSpec((TE, J), lambda i: (i, 0))],
        out_shape=[jax.ShapeDtypeStruct((B, J), _F32)],
        compiler_params=pltpu.CompilerParams(
            dimension_semantics=("arbitrary",)),
    )(obs_t, cmdn, wn, W1f, b1f, c['ex_W2'], c['ex_b2'],
      c['ex_W3'], c['ex_b3'])[0]

    TF = 512
    feats, log_std, log_sig = pl.pallas_call(
        _fn_body,
        grid=(B // TF,),
        in_specs=[
            pl.BlockSpec((TF, OBS), lambda i: (i, 0)),
            pl.BlockSpec((TF, LAT), lambda i: (i, 0)),
            pl.BlockSpec((TF, 1), lambda i: (i, 0)),
            _full_spec((OBS + LAT, HID)), _full_spec((1, HID)),
            _full_spec((1, HID)), _full_spec((1, HID)),
            _full_spec((HID, HID)), _full_spec((1, HID)),
            _full_spec((1, HID)), _full_spec((1, HID)),
            _full_spec((HID, HID)), _full_spec((1, HID)),
            _full_spec((HID, J)), _full_spec((1, J)),
            _full_spec((HID, POS)), _full_spec((1, POS)),
        ],
        out_specs=[
            pl.BlockSpec((TF, HID), lambda i: (i, 0)),
            pl.BlockSpec((TF, J), lambda i: (i, 0)),
            pl.BlockSpec((TF, POS), lambda i: (i, 0)),
        ],
        out_shape=[
            jax.ShapeDtypeStruct((B, HID), _F32),
            jax.ShapeDtypeStruct((B, J), _F32),
            jax.ShapeDtypeStruct((B, POS), _F32),
        ],
        compiler_params=pltpu.CompilerParams(
            dimension_semantics=("arbitrary",)),
    )(obs_t, z, mask_t,
      p['fn_W1'], r2(p['fn_b1']), r2(p['fn_g1']), r2(p['fn_be1']),
      p['fn_W2'], r2(p['fn_b2']), r2(p['fn_g2']), r2(p['fn_be2']),
      p['ls_W1'], r2(p['ls_b1']), p['ls_W2'], r2(p['ls_b2']),
      p['vh_W'], r2(p['vh_b']))

    return (mu_mix, mu_mix, log_std, log_sig, feats)
